# Initial kernel scaffold; baseline (speedup 1.0000x reference)
import jax
import jax.numpy as jnp
from jax.experimental import pallas as pl


def kernel(nms_reg, nms_cls, rcnn_reg, rcnn_cls, bboxes, classes):
    raise NotImplementedError("write your pallas kernel here")


# speedup vs baseline: 31.1910x; 31.1910x over previous
import functools

import jax
import jax.numpy as jnp
from jax.experimental import pallas as pl
from jax.experimental.pallas import tpu as pltpu

TOP, LEFT, BOTTOM, RIGHT = 0, 1, 2, 3
REDUCTION = 16.0
B, T, R, C = 8, 100, 5000, 81
RP = 5120
SUB = 40
NPOS_CAP = 64 * B // 4
NNEG_CAP = 64 * B * 3 // 4


def _lane_inclusive_cumsum(m):
    c = m
    for s in (1, 2, 4, 8, 16, 32, 64):
        c = c + jnp.concatenate([jnp.zeros((SUB, s), jnp.float32), c[:, :-s]], axis=1)
    row_tot = c[:, 127:128]
    rt = row_tot
    for s in (1, 2, 4, 8, 16, 32):
        rt = rt + jnp.concatenate([jnp.zeros((s, 1), jnp.float32), rt[:-s, :]], axis=0)
    return c + (rt - row_tot)


def _loss_kernel(nms_ref, reg_ref, cls_ref, bb_ref, cl_ref,
                 cls_out, reg_out, acc_out, acc_ref):
    b = pl.program_id(0)

    @pl.when(b == 0)
    def _init():
        acc_ref[0] = 0.0
        acc_ref[1] = 0.0
        acc_ref[2] = 0.0
        acc_ref[3] = 0.0
        acc_ref[4] = 0.0

    a_t = nms_ref[0, TOP]
    a_l = nms_ref[0, LEFT]
    a_b = nms_ref[0, BOTTOM]
    a_r = nms_ref[0, RIGHT]
    area_a = jnp.maximum(a_b - a_t, 0.0) * jnp.maximum(a_r - a_l, 0.0)

    def iou_step(t, carry):
        best_iou, best_cls, bb_t, bb_l, bb_b, bb_r = carry
        g_t = bb_ref[0, t, TOP]
        g_l = bb_ref[0, t, LEFT]
        g_b = bb_ref[0, t, BOTTOM]
        g_r = bb_ref[0, t, RIGHT]
        area_b = jnp.maximum(g_b - g_t, 0.0) * jnp.maximum(g_r - g_l, 0.0)
        it = jnp.maximum(a_t, g_t)
        il = jnp.maximum(a_l, g_l)
        ib = jnp.minimum(a_b, g_b)
        ir = jnp.minimum(a_r, g_r)
        inter = jnp.maximum(ib - it, 0.0) * jnp.maximum(ir - il, 0.0)
        union = area_a + area_b - inter
        iou = inter / jnp.maximum(union, 1e-8)
        upd = iou > best_iou
        return (jnp.where(upd, iou, best_iou),
                jnp.where(upd, cl_ref[0, 0, t], best_cls),
                jnp.where(upd, g_t, bb_t),
                jnp.where(upd, g_l, bb_l),
                jnp.where(upd, g_b, bb_b),
                jnp.where(upd, g_r, bb_r))

    init = (jnp.full((SUB, 128), -1.0, jnp.float32),
            jnp.zeros((SUB, 128), jnp.int32),
            jnp.zeros((SUB, 128), jnp.float32),
            jnp.zeros((SUB, 128), jnp.float32),
            jnp.zeros((SUB, 128), jnp.float32),
            jnp.zeros((SUB, 128), jnp.float32))
    best_iou, best_cls, bb_t, bb_l, bb_b, bb_r = jax.lax.fori_loop(
        0, T, iou_step, init)

    row = jax.lax.broadcasted_iota(jnp.int32, (SUB, 128), 0)
    col = jax.lax.broadcasted_iota(jnp.int32, (SUB, 128), 1)
    valid = (row * 128 + col) < R
    is_pos = best_iou > 0.5
    m_pos = (is_pos & valid).astype(jnp.float32)
    m_neg = ((~is_pos) & valid).astype(jnp.float32)

    cum_pos = _lane_inclusive_cumsum(m_pos)
    cum_neg = _lane_inclusive_cumsum(m_neg)
    rank_pos = acc_ref[0] + cum_pos - m_pos
    rank_neg = acc_ref[1] + cum_neg - m_neg
    sel_pos = m_pos * (rank_pos < NPOS_CAP).astype(jnp.float32)
    sel_neg = m_neg * (rank_neg < NNEG_CAP).astype(jnp.float32)

    x = cls_ref[0]
    mx = jnp.max(x, axis=-1)
    lse = mx + jnp.log(jnp.sum(jnp.exp(x - mx[..., None]), axis=-1))
    ci = jax.lax.broadcasted_iota(jnp.int32, (SUB, 128, C), 2)
    x_cls = jnp.sum(jnp.where(ci == best_cls[..., None], x, 0.0), axis=-1)
    am = jnp.min(jnp.where(x == mx[..., None], ci, C), axis=-1)

    cls_sum = (jnp.sum(sel_pos * (x_cls - lse)) +
               jnp.sum(sel_neg * (x[..., 0] - lse)))
    acc_sum = (jnp.sum(sel_pos * (am == best_cls).astype(jnp.float32)) +
               jnp.sum(sel_neg * (am == 0).astype(jnp.float32)))

    reg_sum = jnp.float32(0.0)
    for c, bbc, rnd in ((TOP, bb_t, 0), (LEFT, bb_l, 0),
                        (BOTTOM, bb_b, 1), (RIGHT, bb_r, 1)):
        nv = nms_ref[0, c] * REDUCTION
        rounded = (jnp.ceil(nv) if rnd else jnp.floor(nv)) / REDUCTION
        d = jnp.abs(reg_ref[0, c] - (bbc - rounded))
        term = jnp.where(d < 1.0, 0.5 * d * d, d - 0.5)
        reg_sum = reg_sum + jnp.sum(sel_pos * term)

    acc_ref[2] = acc_ref[2] + cls_sum
    acc_ref[3] = acc_ref[3] + acc_sum
    acc_ref[4] = acc_ref[4] + reg_sum
    acc_ref[0] = acc_ref[0] + jnp.sum(m_pos)
    acc_ref[1] = acc_ref[1] + jnp.sum(m_neg)

    @pl.when(b == B - 1)
    def _fini():
        n_pos = jnp.minimum(acc_ref[0], float(NPOS_CAP))
        n_sel = n_pos + jnp.minimum(acc_ref[1], float(NNEG_CAP))
        cls_out[0, 0] = -acc_ref[2] / n_sel
        acc_out[0, 0] = acc_ref[3] / n_sel
        rl = acc_ref[4] / jnp.maximum(n_pos, 1.0) / 4.0
        reg_out[0, 0] = jnp.where(n_pos > 0.0, rl, 0.0)


@jax.jit
def kernel(nms_reg, nms_cls, rcnn_reg, rcnn_cls, bboxes, classes):
    del nms_cls
    pad = ((0, 0), (0, 0), (0, RP - R))
    nms_t = jnp.pad(jnp.transpose(nms_reg, (0, 2, 1)), pad).reshape(B, 4, SUB, 128)
    reg_t = jnp.pad(jnp.transpose(rcnn_reg, (0, 2, 1)), pad).reshape(B, 4, SUB, 128)
    cls_p = jnp.pad(rcnn_cls, ((0, 0), (0, RP - R), (0, 0))).reshape(B, SUB, 128, C)

    out = pl.pallas_call(
        _loss_kernel,
        grid=(B,),
        in_specs=[
            pl.BlockSpec((1, 4, SUB, 128), lambda b: (b, 0, 0, 0)),
            pl.BlockSpec((1, 4, SUB, 128), lambda b: (b, 0, 0, 0)),
            pl.BlockSpec((1, SUB, 128, C), lambda b: (b, 0, 0, 0)),
            pl.BlockSpec((1, T, 4), lambda b: (b, 0, 0), memory_space=pltpu.SMEM),
            pl.BlockSpec((1, 1, T), lambda b: (b, 0, 0), memory_space=pltpu.SMEM),
        ],
        out_specs=[
            pl.BlockSpec((1, 1), lambda b: (0, 0), memory_space=pltpu.SMEM),
            pl.BlockSpec((1, 1), lambda b: (0, 0), memory_space=pltpu.SMEM),
            pl.BlockSpec((1, 1), lambda b: (0, 0), memory_space=pltpu.SMEM),
        ],
        out_shape=[jax.ShapeDtypeStruct((1, 1), jnp.float32)] * 3,
        scratch_shapes=[pltpu.SMEM((8,), jnp.float32)],
    )(nms_t, reg_t, cls_p, bboxes, classes.reshape(B, 1, T))
    return (out[0].reshape(1), out[1].reshape(1), out[2].reshape(1))
